# Initial kernel scaffold; baseline (speedup 1.0000x reference)
import functools

import numpy as np
import jax
import jax.numpy as jnp
from jax import lax
from jax.experimental import pallas as pl
from jax.experimental.pallas import tpu as pltpu
from jax.experimental.pallas import tpu_sc as plsc

_N1 = 256
_LS = 65536
_W = 1024
_NC = 2
_NS = 16
_NW = _NC * _NS
_CW = 128

_PREC = lax.Precision.HIGHEST


@functools.lru_cache(maxsize=None)
def _dft_consts():
    k = np.arange(_N1, dtype=np.float64)
    ang = -2.0 * np.pi / _N1 * np.outer(k, k)
    dr = np.cos(ang).astype(np.float32)
    di = np.sin(ang).astype(np.float32)
    angt = -2.0 * np.pi / _LS * np.outer(k, k)
    tr = np.cos(angt).astype(np.float32)
    ti = np.sin(angt).astype(np.float32)
    t = np.arange(_W, dtype=np.float64)
    angm = 2.0 * np.pi / _W * np.outer(t, t)
    mr = (np.cos(angm) / _W).astype(np.float32)
    mi = (np.sin(angm) / _W).astype(np.float32)
    return dr, di, tr, ti, mr, mi


def _fft_body(x_ref, dr_ref, di_ref, tr_ref, ti_ref, er_ref, ei_ref):
    a2 = x_ref[0]
    dr = dr_ref[...]
    di = di_ref[...]
    gr = jnp.dot(dr, a2, precision=_PREC, preferred_element_type=jnp.float32)
    gi = jnp.dot(di, a2, precision=_PREC, preferred_element_type=jnp.float32)
    tr = tr_ref[...]
    ti = ti_ref[...]
    hr = gr * tr - gi * ti
    hi = gr * ti + gi * tr
    er_ref[0] = (jnp.dot(hr, dr, precision=_PREC, preferred_element_type=jnp.float32)
                 - jnp.dot(hi, di, precision=_PREC, preferred_element_type=jnp.float32))
    ei_ref[0] = (jnp.dot(hr, di, precision=_PREC, preferred_element_type=jnp.float32)
                 + jnp.dot(hi, dr, precision=_PREC, preferred_element_type=jnp.float32))


def _idft_body(gre_ref, gim_ref, g_ref, mr_ref, mi_ref, or_ref, oi_ref):
    pr = gre_ref[0] * g_ref[...]
    pi = gim_ref[0] * g_ref[...]
    mr = mr_ref[...]
    mi = mi_ref[...]
    or_ref[0] = (jnp.dot(pr, mr, precision=_PREC, preferred_element_type=jnp.float32)
                 - jnp.dot(pi, mi, precision=_PREC, preferred_element_type=jnp.float32))
    oi_ref[0] = (jnp.dot(pr, mi, precision=_PREC, preferred_element_type=jnp.float32)
                 + jnp.dot(pi, mr, precision=_PREC, preferred_element_type=jnp.float32))


def _make_sc_gather(n_chunks, width):
    mesh = plsc.VectorSubcoreMesh(core_axis_name="c", subcore_axis_name="s",
                                  num_cores=_NC, num_subcores=_NS)

    @functools.partial(
        pl.kernel,
        out_type=jax.ShapeDtypeStruct((_NW, n_chunks, _CW, width), jnp.float32),
        mesh=mesh,
        scratch_types=[
            pltpu.VMEM((n_chunks, _CW), jnp.int32),
            pltpu.VMEM((_CW, width), jnp.float32),
            pltpu.SemaphoreType.DMA,
        ],
    )
    def sc_gather(table_hbm, ix_hbm, out_hbm, idx_v, rows_v, sem):
        wid = lax.axis_index("s") * _NC + lax.axis_index("c")
        pltpu.sync_copy(ix_hbm.at[wid], idx_v)
        for k in range(n_chunks):
            pltpu.async_copy(table_hbm.at[idx_v.at[k]], rows_v, sem).wait()
            pltpu.sync_copy(rows_v, out_hbm.at[wid, k])

    return sc_gather


def kernel(x, g, win_ix):
    b, c, ls = x.shape
    f, w = g.shape
    bc = b * c
    assert ls == _LS and w == _W

    dr, di, tr, ti, mr, mi = _dft_consts()

    x3 = x.reshape(bc, _N1, _N1)
    full = pl.BlockSpec((_N1, _N1), lambda i: (0, 0))
    ere, eim = pl.pallas_call(
        _fft_body,
        grid=(bc,),
        in_specs=[pl.BlockSpec((1, _N1, _N1), lambda i: (i, 0, 0)),
                  full, full, full, full],
        out_specs=[pl.BlockSpec((1, _N1, _N1), lambda i: (i, 0, 0))] * 2,
        out_shape=[jax.ShapeDtypeStruct((bc, _N1, _N1), jnp.float32)] * 2,
    )(x3, dr, di, tr, ti)

    table = jnp.stack([ere, eim], axis=-1).transpose(2, 1, 0, 3).reshape(ls, bc * 2)

    n_idx = f * w
    per_w = n_idx // _NW
    n_chunks = per_w // _CW
    ix = win_ix.astype(jnp.int32).reshape(_NW, n_chunks, _CW)
    gathered = _make_sc_gather(n_chunks, bc * 2)(table, ix)

    gt = gathered.reshape(n_idx, bc, 2).transpose(1, 2, 0).reshape(bc, 2, f, w)
    gre, gim = gt[:, 0], gt[:, 1]

    fullg = pl.BlockSpec((f, w), lambda i: (0, 0))
    fullm = pl.BlockSpec((w, w), lambda i: (0, 0))
    ore, oim = pl.pallas_call(
        _idft_body,
        grid=(bc,),
        in_specs=[pl.BlockSpec((1, f, w), lambda i: (i, 0, 0)),
                  pl.BlockSpec((1, f, w), lambda i: (i, 0, 0)),
                  fullg, fullm, fullm],
        out_specs=[pl.BlockSpec((1, f, w), lambda i: (i, 0, 0))] * 2,
        out_shape=[jax.ShapeDtypeStruct((bc, f, w), jnp.float32)] * 2,
    )(gre, gim, g, mr, mi)

    return jnp.stack([ore, oim], axis=-1).reshape(b, c, f, w, 2)


# speedup vs baseline: 9.5819x; 9.5819x over previous
import functools

import numpy as np
import jax
import jax.numpy as jnp
from jax import lax
from jax.experimental import pallas as pl
from jax.experimental.pallas import tpu as pltpu
from jax.experimental.pallas import tpu_sc as plsc

_N1 = 256
_LS = 65536
_W = 1024
_NC = 2
_NS = 16
_NW = _NC * _NS
_CW = 128

_PREC = lax.Precision.HIGHEST


@functools.lru_cache(maxsize=None)
def _dft_consts():
    k = np.arange(_N1, dtype=np.float64)
    ang = -2.0 * np.pi / _N1 * np.outer(k, k)
    dr = np.cos(ang).astype(np.float32)
    di = np.sin(ang).astype(np.float32)
    angt = -2.0 * np.pi / _LS * np.outer(k, k)
    tr = np.cos(angt).astype(np.float32)
    ti = np.sin(angt).astype(np.float32)
    t = np.arange(_W, dtype=np.float64)
    angm = 2.0 * np.pi / _W * np.outer(t, t)
    mr = (np.cos(angm) / _W).astype(np.float32)
    mi = (np.sin(angm) / _W).astype(np.float32)
    return dr, di, tr, ti, mr, mi


def _fft_body(x_ref, dr_ref, di_ref, tr_ref, ti_ref, er_ref, ei_ref):
    a2 = x_ref[0]
    dr = dr_ref[...]
    di = di_ref[...]
    gr = jnp.dot(dr, a2, precision=_PREC, preferred_element_type=jnp.float32)
    gi = jnp.dot(di, a2, precision=_PREC, preferred_element_type=jnp.float32)
    tr = tr_ref[...]
    ti = ti_ref[...]
    hr = gr * tr - gi * ti
    hi = gr * ti + gi * tr
    er_ref[0] = (jnp.dot(hr, dr, precision=_PREC, preferred_element_type=jnp.float32)
                 - jnp.dot(hi, di, precision=_PREC, preferred_element_type=jnp.float32))
    ei_ref[0] = (jnp.dot(hr, di, precision=_PREC, preferred_element_type=jnp.float32)
                 + jnp.dot(hi, dr, precision=_PREC, preferred_element_type=jnp.float32))


def _idft_body(gre_ref, gim_ref, g_ref, mr_ref, mi_ref, or_ref, oi_ref):
    pr = gre_ref[0] * g_ref[...]
    pi = gim_ref[0] * g_ref[...]
    mr = mr_ref[...]
    mi = mi_ref[...]
    or_ref[0] = (jnp.dot(pr, mr, precision=_PREC, preferred_element_type=jnp.float32)
                 - jnp.dot(pi, mi, precision=_PREC, preferred_element_type=jnp.float32))
    oi_ref[0] = (jnp.dot(pr, mi, precision=_PREC, preferred_element_type=jnp.float32)
                 + jnp.dot(pi, mr, precision=_PREC, preferred_element_type=jnp.float32))


def _make_sc_gather(n_chunks, width):
    mesh = plsc.VectorSubcoreMesh(core_axis_name="c", subcore_axis_name="s",
                                  num_cores=_NC, num_subcores=_NS)

    @functools.partial(
        pl.kernel,
        out_type=jax.ShapeDtypeStruct((_NW, n_chunks, _CW, width), jnp.float32),
        mesh=mesh,
        compiler_params=pltpu.CompilerParams(use_tc_tiling_on_sc=False),
        scratch_types=[
            pltpu.VMEM((n_chunks, _CW), jnp.int32),
            pltpu.VMEM((_CW, width), jnp.float32),
            pltpu.SemaphoreType.DMA,
        ],
    )
    def sc_gather(table_hbm, ix_hbm, out_hbm, idx_v, rows_v, sem):
        wid = lax.axis_index("s") * _NC + lax.axis_index("c")
        pltpu.sync_copy(ix_hbm.at[wid], idx_v)
        for k in range(n_chunks):
            pltpu.async_copy(table_hbm.at[idx_v.at[k]], rows_v, sem).wait()
            pltpu.sync_copy(rows_v, out_hbm.at[wid, k])

    return sc_gather


def kernel(x, g, win_ix):
    b, c, ls = x.shape
    f, w = g.shape
    bc = b * c
    assert ls == _LS and w == _W

    dr, di, tr, ti, mr, mi = _dft_consts()

    x3 = x.reshape(bc, _N1, _N1)
    full = pl.BlockSpec((_N1, _N1), lambda i: (0, 0))
    ere, eim = pl.pallas_call(
        _fft_body,
        grid=(bc,),
        in_specs=[pl.BlockSpec((1, _N1, _N1), lambda i: (i, 0, 0)),
                  full, full, full, full],
        out_specs=[pl.BlockSpec((1, _N1, _N1), lambda i: (i, 0, 0))] * 2,
        out_shape=[jax.ShapeDtypeStruct((bc, _N1, _N1), jnp.float32)] * 2,
    )(x3, dr, di, tr, ti)

    table = jnp.stack([ere, eim], axis=-1).transpose(2, 1, 0, 3).reshape(ls, bc * 2)

    n_idx = f * w
    per_w = n_idx // _NW
    n_chunks = per_w // _CW
    ix = win_ix.astype(jnp.int32).reshape(_NW, n_chunks, _CW)
    gathered = _make_sc_gather(n_chunks, bc * 2)(table, ix)

    gt = gathered.reshape(n_idx, bc, 2).transpose(1, 2, 0).reshape(bc, 2, f, w)
    gre, gim = gt[:, 0], gt[:, 1]

    fullg = pl.BlockSpec((f, w), lambda i: (0, 0))
    fullm = pl.BlockSpec((w, w), lambda i: (0, 0))
    ore, oim = pl.pallas_call(
        _idft_body,
        grid=(bc,),
        in_specs=[pl.BlockSpec((1, f, w), lambda i: (i, 0, 0)),
                  pl.BlockSpec((1, f, w), lambda i: (i, 0, 0)),
                  fullg, fullm, fullm],
        out_specs=[pl.BlockSpec((1, f, w), lambda i: (i, 0, 0))] * 2,
        out_shape=[jax.ShapeDtypeStruct((bc, f, w), jnp.float32)] * 2,
    )(gre, gim, g, mr, mi)

    return jnp.stack([ore, oim], axis=-1).reshape(b, c, f, w, 2)


# speedup vs baseline: 19.5130x; 2.0365x over previous
import functools

import numpy as np
import jax
import jax.numpy as jnp
from jax import lax
from jax.experimental import pallas as pl
from jax.experimental.pallas import tpu as pltpu
from jax.experimental.pallas import tpu_sc as plsc

_N1 = 256
_LS = 65536
_W = 1024
_NC = 2
_NS = 16
_NW = _NC * _NS
_CW = 128

_PREC = lax.Precision.HIGHEST


@functools.lru_cache(maxsize=None)
def _dft_consts():
    k = np.arange(_N1, dtype=np.float64)
    ang = -2.0 * np.pi / _N1 * np.outer(k, k)
    dr = np.cos(ang).astype(np.float32)
    di = np.sin(ang).astype(np.float32)
    angt = -2.0 * np.pi / _LS * np.outer(k, k)
    tr = np.cos(angt).astype(np.float32)
    ti = np.sin(angt).astype(np.float32)
    t = np.arange(_W, dtype=np.float64)
    angm = 2.0 * np.pi / _W * np.outer(t, t)
    sgn = np.where(t % 2 == 0, 1.0, -1.0)[None, :]
    mr = (np.cos(angm) * sgn / _W).astype(np.float32)
    mi = (np.sin(angm) * sgn / _W).astype(np.float32)
    return dr, di, tr, ti, mr, mi


def _fft_body(x_ref, dr_ref, di_ref, tr_ref, ti_ref, er_ref, ei_ref):
    a2 = x_ref[0]
    dr = dr_ref[...]
    di = di_ref[...]
    gr = jnp.dot(dr, a2, precision=_PREC, preferred_element_type=jnp.float32)
    gi = jnp.dot(di, a2, precision=_PREC, preferred_element_type=jnp.float32)
    tr = tr_ref[...]
    ti = ti_ref[...]
    hr = gr * tr - gi * ti
    hi = gr * ti + gi * tr
    er_ref[0] = (jnp.dot(hr, dr, precision=_PREC, preferred_element_type=jnp.float32)
                 - jnp.dot(hi, di, precision=_PREC, preferred_element_type=jnp.float32))
    ei_ref[0] = (jnp.dot(hr, di, precision=_PREC, preferred_element_type=jnp.float32)
                 + jnp.dot(hi, dr, precision=_PREC, preferred_element_type=jnp.float32))


def _idft_body(gre_ref, gim_ref, g_ref, mr_ref, mi_ref, or_ref, oi_ref):
    pr = gre_ref[0] * g_ref[...]
    pi = gim_ref[0] * g_ref[...]
    mr = mr_ref[...]
    mi = mi_ref[...]
    or_ref[0] = (jnp.dot(pr, mr, precision=_PREC, preferred_element_type=jnp.float32)
                 - jnp.dot(pi, mi, precision=_PREC, preferred_element_type=jnp.float32))
    oi_ref[0] = (jnp.dot(pr, mi, precision=_PREC, preferred_element_type=jnp.float32)
                 + jnp.dot(pi, mr, precision=_PREC, preferred_element_type=jnp.float32))


def _make_sc_gather(f, width):
    bands_per_w = f // _NW
    mesh = plsc.VectorSubcoreMesh(core_axis_name="c", subcore_axis_name="s",
                                  num_cores=_NC, num_subcores=_NS)

    @functools.partial(
        pl.kernel,
        out_type=jax.ShapeDtypeStruct((f, _W, width), jnp.float32),
        mesh=mesh,
        compiler_params=pltpu.CompilerParams(use_tc_tiling_on_sc=False,
                                             needs_layout_passes=False),
        scratch_types=[
            [pltpu.VMEM((16,), jnp.int32) for _ in range(3)],
            [pltpu.VMEM((_W, width), jnp.float32) for _ in range(3)],
            pltpu.SemaphoreType.DMA,
        ],
    )
    def sc_gather(table_hbm, ix_hbm, out_hbm, tp_vs, win_vs, sem):
        wid = lax.axis_index("s") * _NC + lax.axis_index("c")
        lane = lax.iota(jnp.int32, 16)
        for j in range(bands_per_w):
            pltpu.sync_copy(ix_hbm.at[wid * bands_per_w + j, pl.ds(0, 16)],
                            tp_vs[j])
        copies = []
        for j in range(bands_per_w):
            tp = jnp.max(jnp.where(lane == 0, tp_vs[j][...], 0))
            copies.append(
                pltpu.async_copy(table_hbm.at[pl.ds(tp, _W)], win_vs[j], sem))
        for j in range(bands_per_w):
            copies[j].wait()
            pltpu.sync_copy(win_vs[j], out_hbm.at[wid * bands_per_w + j])

    return sc_gather


def kernel(x, g, win_ix):
    b, c, ls = x.shape
    f, w = g.shape
    bc = b * c
    assert ls == _LS and w == _W

    dr, di, tr, ti, mr, mi = _dft_consts()

    x3 = x.reshape(bc, _N1, _N1)
    full = pl.BlockSpec((_N1, _N1), lambda i: (0, 0))
    ere, eim = pl.pallas_call(
        _fft_body,
        grid=(bc,),
        in_specs=[pl.BlockSpec((1, _N1, _N1), lambda i: (i, 0, 0)),
                  full, full, full, full],
        out_specs=[pl.BlockSpec((1, _N1, _N1), lambda i: (i, 0, 0))] * 2,
        out_shape=[jax.ShapeDtypeStruct((bc, _N1, _N1), jnp.float32)] * 2,
    )(x3, dr, di, tr, ti)

    table = jnp.stack([ere, eim], axis=-1).transpose(2, 1, 0, 3).reshape(ls, bc * 2)
    tablep = jnp.concatenate(
        [table[ls - _W // 2:], table, table[:_W // 2]], axis=0)

    ix = win_ix.astype(jnp.int32)
    gathered = _make_sc_gather(f, bc * 2)(tablep, ix)

    gt = gathered.reshape(f, w, bc, 2).transpose(2, 3, 0, 1)
    gre, gim = gt[:, 0], gt[:, 1]
    gs = jnp.roll(g, -(_W // 2), axis=1)

    fullg = pl.BlockSpec((f, w), lambda i: (0, 0))
    fullm = pl.BlockSpec((w, w), lambda i: (0, 0))
    ore, oim = pl.pallas_call(
        _idft_body,
        grid=(bc,),
        in_specs=[pl.BlockSpec((1, f, w), lambda i: (i, 0, 0)),
                  pl.BlockSpec((1, f, w), lambda i: (i, 0, 0)),
                  fullg, fullm, fullm],
        out_specs=[pl.BlockSpec((1, f, w), lambda i: (i, 0, 0))] * 2,
        out_shape=[jax.ShapeDtypeStruct((bc, f, w), jnp.float32)] * 2,
    )(gre, gim, gs, mr, mi)

    return jnp.stack([ore, oim], axis=-1).reshape(b, c, f, w, 2)


# speedup vs baseline: 27.1264x; 1.3902x over previous
import functools

import numpy as np
import jax
import jax.numpy as jnp
from jax import lax
from jax.experimental import pallas as pl
from jax.experimental.pallas import tpu as pltpu
from jax.experimental.pallas import tpu_sc as plsc

_N1 = 256
_LS = 65536
_W = 1024
_NC = 2
_NS = 16
_NW = _NC * _NS
_CW = 128

_PREC = lax.Precision.DEFAULT


@functools.lru_cache(maxsize=None)
def _dft_consts():
    k = np.arange(_N1, dtype=np.float64)
    ang = -2.0 * np.pi / _N1 * np.outer(k, k)
    dr = np.cos(ang).astype(np.float32)
    di = np.sin(ang).astype(np.float32)
    angt = -2.0 * np.pi / _LS * np.outer(k, k)
    tr = np.cos(angt).astype(np.float32)
    ti = np.sin(angt).astype(np.float32)
    t = np.arange(_W, dtype=np.float64)
    angm = 2.0 * np.pi / _W * np.outer(t, t)
    sgn = np.where(t % 2 == 0, 1.0, -1.0)[None, :]
    mr = (np.cos(angm) * sgn / _W).astype(np.float32)
    mi = (np.sin(angm) * sgn / _W).astype(np.float32)
    return dr, di, tr, ti, mr, mi


def _fft_body(x_ref, dr_ref, di_ref, tr_ref, ti_ref, er_ref, ei_ref):
    a2 = x_ref[0]
    dr = dr_ref[...]
    di = di_ref[...]
    gr = jnp.dot(dr, a2, precision=_PREC, preferred_element_type=jnp.float32)
    gi = jnp.dot(di, a2, precision=_PREC, preferred_element_type=jnp.float32)
    tr = tr_ref[...]
    ti = ti_ref[...]
    hr = gr * tr - gi * ti
    hi = gr * ti + gi * tr
    er_ref[0] = (jnp.dot(hr, dr, precision=_PREC, preferred_element_type=jnp.float32)
                 - jnp.dot(hi, di, precision=_PREC, preferred_element_type=jnp.float32))
    ei_ref[0] = (jnp.dot(hr, di, precision=_PREC, preferred_element_type=jnp.float32)
                 + jnp.dot(hi, dr, precision=_PREC, preferred_element_type=jnp.float32))


def _idft_body(gre_ref, gim_ref, g_ref, mr_ref, mi_ref, mrpmi_ref, or_ref, oi_ref):
    pr = gre_ref[0] * g_ref[...]
    pi = gim_ref[0] * g_ref[...]
    t1 = jnp.dot(pr, mr_ref[...], precision=_PREC, preferred_element_type=jnp.float32)
    t2 = jnp.dot(pi, mi_ref[...], precision=_PREC, preferred_element_type=jnp.float32)
    t3 = jnp.dot(pr + pi, mrpmi_ref[...], precision=_PREC,
                 preferred_element_type=jnp.float32)
    or_ref[0] = t1 - t2
    oi_ref[0] = t3 - t1 - t2


def _make_sc_gather(f, width):
    bands_per_w = f // _NW
    mesh = plsc.VectorSubcoreMesh(core_axis_name="c", subcore_axis_name="s",
                                  num_cores=_NC, num_subcores=_NS)

    @functools.partial(
        pl.kernel,
        out_type=jax.ShapeDtypeStruct((f, _W, width), jnp.float32),
        mesh=mesh,
        compiler_params=pltpu.CompilerParams(use_tc_tiling_on_sc=False,
                                             needs_layout_passes=False),
        scratch_types=[
            [pltpu.VMEM((16,), jnp.int32) for _ in range(3)],
            [pltpu.VMEM((_W, width), jnp.float32) for _ in range(3)],
            pltpu.SemaphoreType.DMA,
        ],
    )
    def sc_gather(table_hbm, ix_hbm, out_hbm, tp_vs, win_vs, sem):
        wid = lax.axis_index("s") * _NC + lax.axis_index("c")
        lane = lax.iota(jnp.int32, 16)
        for j in range(bands_per_w):
            pltpu.sync_copy(ix_hbm.at[wid * bands_per_w + j, pl.ds(0, 16)],
                            tp_vs[j])
        copies = []
        for j in range(bands_per_w):
            tp = jnp.max(jnp.where(lane == 0, tp_vs[j][...], 0))
            copies.append(
                pltpu.async_copy(table_hbm.at[pl.ds(tp, _W)], win_vs[j], sem))
        for j in range(bands_per_w):
            copies[j].wait()
            pltpu.sync_copy(win_vs[j], out_hbm.at[wid * bands_per_w + j])

    return sc_gather


def kernel(x, g, win_ix):
    b, c, ls = x.shape
    f, w = g.shape
    bc = b * c
    assert ls == _LS and w == _W

    dr, di, tr, ti, mr, mi = _dft_consts()

    x3 = x.reshape(bc, _N1, _N1)
    full = pl.BlockSpec((_N1, _N1), lambda i: (0, 0))
    ere, eim = pl.pallas_call(
        _fft_body,
        grid=(bc,),
        in_specs=[pl.BlockSpec((1, _N1, _N1), lambda i: (i, 0, 0)),
                  full, full, full, full],
        out_specs=[pl.BlockSpec((1, _N1, _N1), lambda i: (i, 0, 0))] * 2,
        out_shape=[jax.ShapeDtypeStruct((bc, _N1, _N1), jnp.float32)] * 2,
    )(x3, dr, di, tr, ti)

    table = jnp.stack([ere, eim], axis=-1).transpose(2, 1, 0, 3).reshape(ls, bc * 2)
    tablep = jnp.concatenate(
        [table[ls - _W // 2:], table, table[:_W // 2]], axis=0)

    ix = win_ix.astype(jnp.int32)
    gathered = _make_sc_gather(f, bc * 2)(tablep, ix)

    gt = gathered.reshape(f, w, bc, 2).transpose(2, 3, 0, 1)
    gre, gim = gt[:, 0], gt[:, 1]
    gs = jnp.roll(g, -(_W // 2), axis=1)

    fullg = pl.BlockSpec((f, w), lambda i: (0, 0))
    fullm = pl.BlockSpec((w, w), lambda i: (0, 0))
    ore, oim = pl.pallas_call(
        _idft_body,
        grid=(bc,),
        in_specs=[pl.BlockSpec((1, f, w), lambda i: (i, 0, 0)),
                  pl.BlockSpec((1, f, w), lambda i: (i, 0, 0)),
                  fullg, fullm, fullm, fullm],
        out_specs=[pl.BlockSpec((1, f, w), lambda i: (i, 0, 0))] * 2,
        out_shape=[jax.ShapeDtypeStruct((bc, f, w), jnp.float32)] * 2,
    )(gre, gim, gs, mr, mi, mr + mi)

    return jnp.stack([ore, oim], axis=-1).reshape(b, c, f, w, 2)
